# Initial kernel scaffold; baseline (speedup 1.0000x reference)
import jax
import jax.numpy as jnp
from jax.experimental import pallas as pl


def kernel(in_feat, edge_index, W1, b1, W2, b2, Wfc1, Wfc2):
    raise NotImplementedError("write your pallas kernel here")


# speedup vs baseline: 7.8140x; 7.8140x over previous
import functools

import jax
import jax.numpy as jnp
from jax import lax
from jax.experimental import pallas as pl
from jax.experimental.pallas import tpu as pltpu
from jax.experimental.pallas import tpu_sc as plsc

_N = 10000
_E = 320000
_D = 128
_NP = 10240
_NC = 2
_NS = 16
_NW = _NC * _NS
_C = 80
_ECH = (_E // _NW) // _C
_DCH = (_E // _NS) // _C
_DW = 16
_RPT = _NP // _NS
_ZR = 128


_IB = 5


_EPW = _E // _NW


def _deg2_body(sidx_hbm, didx_hbm, out_hbm, sidx_v, didx_v, hs_v, hd_v):
    c = lax.axis_index("c")
    s = lax.axis_index("s")
    w = s * _NC + c
    z16 = jnp.zeros((16,), jnp.float32)

    def zero(i, carry):
        hs_v[pl.ds(i * 16, 16)] = z16
        hd_v[pl.ds(i * 16, 16)] = z16
        return carry

    lax.fori_loop(0, _NP // 16, zero, 0)
    pltpu.sync_copy(sidx_hbm.at[w], sidx_v)
    pltpu.sync_copy(didx_hbm.at[w], didx_v)
    ones16 = jnp.ones((16,), jnp.float32)

    def step(j, carry):
        vs = sidx_v[pl.ds(j * 16, 16)]
        plsc.addupdate_scatter(hs_v, [vs], ones16)
        vd = didx_v[pl.ds(j * 16, 16)]
        plsc.addupdate_scatter(hd_v, [vd], ones16)
        return carry

    lax.fori_loop(0, _EPW // 16, step, 0)
    pltpu.sync_copy(hs_v, out_hbm.at[0, w])
    pltpu.sync_copy(hd_v, out_hbm.at[1, w])


def _sc_degrees2(edge_index):
    sidx = edge_index[0].reshape(_NW, _EPW)
    didx = edge_index[1].reshape(_NW, _EPW)
    kern = pl.kernel(
        _deg2_body,
        out_type=jax.ShapeDtypeStruct((2, _NW, _NP), jnp.float32),
        mesh=plsc.VectorSubcoreMesh(core_axis_name="c", subcore_axis_name="s"),
        compiler_params=pltpu.CompilerParams(needs_layout_passes=False),
        scratch_types=[
            pltpu.VMEM((_EPW,), jnp.int32),
            pltpu.VMEM((_EPW,), jnp.int32),
            pltpu.VMEM((_NP,), jnp.float32),
            pltpu.VMEM((_NP,), jnp.float32),
        ],
    )
    return kern(sidx, didx)


def _agg_body(tab_hbm, sidx_hbm, didx_hbm, zeros_hbm, out_hbm,
              sidx_v, didx_v, buf_v, acc_sh, sem):
    c = lax.axis_index("c")
    s = lax.axis_index("s")
    w = s * _NC + c
    pltpu.sync_copy(zeros_hbm, buf_v)
    for k in range(_RPT // _C):
        pltpu.sync_copy(buf_v, acc_sh.at[pl.ds(s * _RPT + k * _C, _C)])
    plsc.subcore_barrier()

    def outer(j, carry):
        pltpu.sync_copy(sidx_hbm.at[w, j], sidx_v)
        pltpu.sync_copy(didx_hbm.at[w, j], didx_v)
        for jj in range(_IB):
            pltpu.async_copy(tab_hbm.at[sidx_v.at[jj]], buf_v, sem).wait()
            pltpu.sync_copy(buf_v, acc_sh.at[didx_v.at[jj]], add=True)
        return carry

    lax.fori_loop(0, _ECH // _IB, outer, 0)
    plsc.subcore_barrier()
    for k in range(_RPT // _C):
        rows = pl.ds(s * _RPT + k * _C, _C)
        pltpu.sync_copy(acc_sh.at[rows], buf_v)
        pltpu.sync_copy(buf_v, out_hbm.at[c, rows])


def _sc_aggregate(table, sidx, didx, zeros_blk):
    kern = pl.kernel(
        _agg_body,
        out_type=jax.ShapeDtypeStruct((_NC, _NP, _D), jnp.float32),
        mesh=plsc.VectorSubcoreMesh(core_axis_name="c", subcore_axis_name="s"),
        scratch_types=[
            pltpu.VMEM((_IB, _C), jnp.int32),
            pltpu.VMEM((_IB, _C), jnp.int32),
            pltpu.VMEM((_C, _D), jnp.float32),
            pltpu.VMEM_SHARED((_NP, _D), jnp.float32),
            pltpu.SemaphoreType.DMA,
        ],
    )
    return kern(table, sidx, didx, zeros_blk)


_RB = 2048


def _norm_from(deg_ref, which):
    ones_w = jnp.ones((_NW, 1), jnp.float32)
    d = lax.dot_general(deg_ref[which], ones_w, (((0,), (0,)), ((), ())),
                        precision=lax.Precision.HIGHEST,
                        preferred_element_type=jnp.float32)
    return lax.rsqrt(jnp.maximum(d, 1.0))


def _xn_body(x_ref, deg_ref, xn_ref):
    xn_ref[...] = x_ref[...] * _norm_from(deg_ref, 0)


def _tc_normalize(x_pad, degs):
    return pl.pallas_call(
        _xn_body,
        grid=(_NP // _RB,),
        in_specs=[
            pl.BlockSpec((_RB, _D), lambda i: (i, 0)),
            pl.BlockSpec((2, _NW, _RB), lambda i: (0, 0, i)),
        ],
        out_specs=pl.BlockSpec((_RB, _D), lambda i: (i, 0)),
        out_shape=jax.ShapeDtypeStruct((_NP, _D), jnp.float32),
    )(x_pad, degs)


def _conv1_body(p_ref, deg_ref, w1_ref, b1_ref, w2_ref, g_ref):
    agg = (p_ref[0] + p_ref[1]) * _norm_from(deg_ref, 1)
    h1 = jnp.maximum(
        jnp.dot(agg, w1_ref[...], preferred_element_type=jnp.float32,
                precision=lax.Precision.HIGHEST) + b1_ref[...], 0.0)
    g_ref[...] = jnp.dot(h1 * _norm_from(deg_ref, 0), w2_ref[...],
                         preferred_element_type=jnp.float32,
                         precision=lax.Precision.HIGHEST)


def _tc_conv1(parts, degs, W1, b1, W2):
    return pl.pallas_call(
        _conv1_body,
        grid=(_NP // _RB,),
        in_specs=[
            pl.BlockSpec((2, _RB, _D), lambda i: (0, i, 0)),
            pl.BlockSpec((2, _NW, _RB), lambda i: (0, 0, i)),
            pl.BlockSpec((_D, 2 * _D), lambda i: (0, 0)),
            pl.BlockSpec((1, 2 * _D), lambda i: (0, 0)),
            pl.BlockSpec((2 * _D, _D), lambda i: (0, 0)),
        ],
        out_specs=pl.BlockSpec((_RB, _D), lambda i: (i, 0)),
        out_shape=jax.ShapeDtypeStruct((_NP, _D), jnp.float32),
    )(parts, degs, W1, b1, W2)


def _conv2_body(p_ref, deg_ref, b2_ref, f1w_ref, f2w_ref,
                h_ref, f1_ref, f2_ref):
    agg = (p_ref[0] + p_ref[1]) * _norm_from(deg_ref, 1)
    h = jnp.maximum(agg + b2_ref[...], 0.0)
    h_ref[...] = h
    f1_ref[...] = jnp.dot(h, f1w_ref[...], preferred_element_type=jnp.float32,
                          precision=lax.Precision.HIGHEST)
    f2_ref[...] = jnp.dot(h, f2w_ref[...], preferred_element_type=jnp.float32,
                          precision=lax.Precision.HIGHEST)


def _tc_conv2(parts, degs, b2, Wfc1T, Wfc2T):
    out_sd = jax.ShapeDtypeStruct((_NP, _D), jnp.float32)
    return pl.pallas_call(
        _conv2_body,
        grid=(_NP // _RB,),
        in_specs=[
            pl.BlockSpec((2, _RB, _D), lambda i: (0, i, 0)),
            pl.BlockSpec((2, _NW, _RB), lambda i: (0, 0, i)),
            pl.BlockSpec((1, _D), lambda i: (0, 0)),
            pl.BlockSpec((_D, _D), lambda i: (0, 0)),
            pl.BlockSpec((_D, _D), lambda i: (0, 0)),
        ],
        out_specs=[pl.BlockSpec((_RB, _D), lambda i: (i, 0))] * 3,
        out_shape=[out_sd, out_sd, out_sd],
    )(parts, degs, b2, Wfc1T, Wfc2T)


def kernel(in_feat, edge_index, W1, b1, W2, b2, Wfc1, Wfc2):
    src = edge_index[0].reshape(_NW, _ECH // _IB, _IB, _C)
    dst = edge_index[1].reshape(_NW, _ECH // _IB, _IB, _C)
    x_pad = jnp.zeros((_NP, _D), jnp.float32).at[:_N].set(in_feat)
    zeros_blk = jnp.zeros((_C, _D), jnp.float32)

    degs = _sc_degrees2(edge_index)
    xn = _tc_normalize(x_pad, degs)
    p1 = _sc_aggregate(xn, src, dst, zeros_blk)
    g1 = _tc_conv1(p1, degs, W1, b1.reshape(1, 2 * _D), W2)
    p2 = _sc_aggregate(g1, src, dst, zeros_blk)
    h, f1, f2 = _tc_conv2(p2, degs, b2.reshape(1, _D),
                          Wfc1.T, Wfc2.T)
    return (h[:_N], f1[:_N], f2[:_N])
